# baseline (speedup 1.0000x reference)
import functools

import jax
import jax.numpy as jnp
from jax import lax
from jax.experimental import pallas as pl
from jax.experimental.pallas import tpu as pltpu
from jax.experimental.pallas import tpu_sc as plsc

EMB_D = 64
USR_D = 16
NEXP = 64
BLK = 4096


def _gate_body(h_ref, u_ref, wg_ref, bg_ref, wb_ref, bb_ref, wl_ref,
               bl_ref, out_ref):
    u = u_ref[...]
    h = h_ref[...]
    gamma = jnp.dot(u, wg_ref[...], preferred_element_type=jnp.float32)
    gamma = gamma + bg_ref[...]
    beta = jnp.dot(u, wb_ref[...], preferred_element_type=jnp.float32)
    beta = beta + bb_ref[...]
    h_t = h * (1.0 + gamma) + beta
    logits = jnp.dot(h_t, wl_ref[...], preferred_element_type=jnp.float32)
    logits = logits + bl_ref[...]

    row = lax.broadcasted_iota(jnp.int32, (NEXP, NEXP), 0)
    col = lax.broadcasted_iota(jnp.int32, (NEXP, NEXP), 1)
    lt = (row <= col).astype(jnp.float32)

    m1 = jnp.max(logits, axis=1, keepdims=True)
    eq1 = logits == m1
    cs1 = jnp.dot(eq1.astype(jnp.float32), lt,
                  preferred_element_type=jnp.float32)
    mask1 = eq1 & (cs1 == 1.0)
    l2 = jnp.where(mask1, -jnp.inf, logits)
    m2 = jnp.max(l2, axis=1, keepdims=True)
    eq2 = l2 == m2
    cs2 = jnp.dot(eq2.astype(jnp.float32), lt,
                  preferred_element_type=jnp.float32)
    mask2 = eq2 & (cs2 == 1.0)

    e = jnp.exp(m2 - m1)
    denom = 1.0 + e
    p1 = 1.0 / denom
    p2 = e / denom
    out_ref[...] = jnp.where(mask1, p1, jnp.where(mask2, p2, 0.0))


_N_TOK = 32768
_SC_ROWS = _N_TOK // 32


@functools.partial(
    pl.kernel,
    out_type=jax.ShapeDtypeStruct((_N_TOK, EMB_D), jnp.float32),
    mesh=plsc.VectorSubcoreMesh(core_axis_name="c", subcore_axis_name="s"),
    scratch_types=[pltpu.VMEM((_SC_ROWS, EMB_D), jnp.float32)],
)
def _sc_copy(h_hbm, out_hbm, buf):
    wid = lax.axis_index("s") * 2 + lax.axis_index("c")
    base = wid * _SC_ROWS
    pltpu.sync_copy(h_hbm.at[pl.ds(base, _SC_ROWS)], buf)
    pltpu.sync_copy(buf, out_hbm.at[pl.ds(base, _SC_ROWS)])


def kernel(h, u, Wg, bg, Wb, bb, Wl, bl):
    half = h.shape[0] // 2
    wa = _half(h[:half], u[:half], Wg, bg, Wb, bb, Wl, bl)
    wb = _half(h[half:], u[half:], Wg, bg, Wb, bb, Wl, bl)
    return jnp.concatenate([wa, wb], axis=0)


def _half(h, u, Wg, bg, Wb, bb, Wl, bl):
    n = h.shape[0]
    grid = (n // BLK,)
    bg2 = bg[None, :]
    bb2 = bb[None, :]
    bl2 = bl[None, :]
    w = pl.pallas_call(
        _gate_body,
        grid=grid,
        in_specs=[
            pl.BlockSpec((BLK, EMB_D), lambda i: (i, 0)),
            pl.BlockSpec((BLK, USR_D), lambda i: (i, 0)),
            pl.BlockSpec((USR_D, EMB_D), lambda i: (0, 0)),
            pl.BlockSpec((1, EMB_D), lambda i: (0, 0)),
            pl.BlockSpec((USR_D, EMB_D), lambda i: (0, 0)),
            pl.BlockSpec((1, EMB_D), lambda i: (0, 0)),
            pl.BlockSpec((EMB_D, NEXP), lambda i: (0, 0)),
            pl.BlockSpec((1, NEXP), lambda i: (0, 0)),
        ],
        out_specs=pl.BlockSpec((BLK, NEXP), lambda i: (i, 0)),
        out_shape=jax.ShapeDtypeStruct((n, NEXP), jnp.float32),
    )(h, u, Wg.T, bg2, Wb.T, bb2, Wl.T, bl2)
    return w


# speedup vs baseline: 1.5736x; 1.5736x over previous
import jax
import jax.numpy as jnp
from jax import lax
from jax.experimental import pallas as pl

EMB_D = 64
USR_D = 16
NEXP = 64
BLK = 4096


def _top2(logits):
    row = lax.broadcasted_iota(jnp.int32, (NEXP, NEXP), 0)
    col = lax.broadcasted_iota(jnp.int32, (NEXP, NEXP), 1)
    lt = (row <= col).astype(jnp.float32)
    m1 = jnp.max(logits, axis=1, keepdims=True)
    eq1 = logits == m1
    cs1 = jnp.dot(eq1.astype(jnp.float32), lt,
                  preferred_element_type=jnp.float32)
    mask1 = eq1 & (cs1 == 1.0)
    l2 = jnp.where(mask1, -jnp.inf, logits)
    m2 = jnp.max(l2, axis=1, keepdims=True)
    eq2 = l2 == m2
    cs2 = jnp.dot(eq2.astype(jnp.float32), lt,
                  preferred_element_type=jnp.float32)
    mask2 = eq2 & (cs2 == 1.0)
    e = jnp.exp(m2 - m1)
    denom = 1.0 + e
    p1 = 1.0 / denom
    p2 = e / denom
    return jnp.where(mask1, p1, jnp.where(mask2, p2, 0.0))


def _gate_half(h, u, wg, bg, wb, bb, wl, bl):
    gamma = jnp.dot(u, wg, preferred_element_type=jnp.float32) + bg
    beta = jnp.dot(u, wb, preferred_element_type=jnp.float32) + bb
    h_t = h * (1.0 + gamma) + beta
    logits = jnp.dot(h_t, wl, preferred_element_type=jnp.float32) + bl
    return _top2(logits)


def _gate_body(hlo_ref, hhi_ref, ulo_ref, uhi_ref, wg_ref, bg_ref, wb_ref,
               bb_ref, wl_ref, bl_ref, out_ref):
    wg = wg_ref[...]
    bg = bg_ref[...]
    wb = wb_ref[...]
    bb = bb_ref[...]
    wl = wl_ref[...]
    bl = bl_ref[...]
    out_ref[:BLK, :] = _gate_half(hlo_ref[...], ulo_ref[...],
                                  wg, bg, wb, bb, wl, bl)
    out_ref[BLK:, :] = _gate_half(hhi_ref[...], uhi_ref[...],
                                  wg, bg, wb, bb, wl, bl)


def kernel(h, u, Wg, bg, Wb, bb, Wl, bl):
    n = h.shape[0]
    grid = (n // (2 * BLK),)
    return pl.pallas_call(
        _gate_body,
        grid=grid,
        in_specs=[
            pl.BlockSpec((BLK, EMB_D), lambda i: (2 * i, 0)),
            pl.BlockSpec((BLK, EMB_D), lambda i: (2 * i + 1, 0)),
            pl.BlockSpec((BLK, USR_D), lambda i: (2 * i, 0)),
            pl.BlockSpec((BLK, USR_D), lambda i: (2 * i + 1, 0)),
            pl.BlockSpec((USR_D, EMB_D), lambda i: (0, 0)),
            pl.BlockSpec((1, EMB_D), lambda i: (0, 0)),
            pl.BlockSpec((USR_D, EMB_D), lambda i: (0, 0)),
            pl.BlockSpec((1, EMB_D), lambda i: (0, 0)),
            pl.BlockSpec((EMB_D, NEXP), lambda i: (0, 0)),
            pl.BlockSpec((1, NEXP), lambda i: (0, 0)),
        ],
        out_specs=pl.BlockSpec((2 * BLK, NEXP), lambda i: (i, 0)),
        out_shape=jax.ShapeDtypeStruct((n, NEXP), jnp.float32),
    )(h, h, u, u, Wg.T, bg[None, :], Wb.T, bb[None, :], Wl.T, bl[None, :])
